# Initial kernel scaffold; baseline (speedup 1.0000x reference)
import jax
import jax.numpy as jnp
from jax.experimental import pallas as pl


def kernel(points, features):
    raise NotImplementedError("write your pallas kernel here")


# speedup vs baseline: 1.0231x; 1.0231x over previous
import functools

import jax
import jax.numpy as jnp
from jax.experimental import pallas as pl

KNN = 16
RB = 256


def _topk_body(srcp_ref, srcpT_ref, idx_ref):
    x = srcp_ref[0]
    y = srcpT_ref[0]
    n = y.shape[1]
    d = -2.0 * jnp.dot(x, y, preferred_element_type=jnp.float32)
    sqr = jnp.sum(x * x, axis=1, keepdims=True)
    sqc = jnp.sum(y * y, axis=0, keepdims=True)
    d = d + sqr + sqc
    d = jnp.maximum(d, 1e-12)
    colid = jax.lax.broadcasted_iota(jnp.int32, (RB, n), 1)
    cols = []
    for _ in range(KNN):
        m = jnp.min(d, axis=1, keepdims=True)
        a = jnp.min(jnp.where(d == m, colid, n), axis=1)
        cols.append(a)
        d = jnp.where(colid == a[:, None], jnp.inf, d)
    idx_ref[0] = jnp.stack(cols, axis=1)


def _knn_indices(points):
    B, _, N = points.shape
    src = jnp.transpose(points, (0, 2, 1))
    srcp = jnp.pad(src, ((0, 0), (0, 0), (0, 5)))
    srcpT = jnp.transpose(srcp, (0, 2, 1))
    grid = (B, N // RB)
    return pl.pallas_call(
        _topk_body,
        grid=grid,
        in_specs=[
            pl.BlockSpec((1, RB, 8), lambda b, i: (b, i, 0)),
            pl.BlockSpec((1, 8, N), lambda b, i: (b, 0, 0)),
        ],
        out_specs=pl.BlockSpec((1, RB, KNN), lambda b, i: (b, i, 0)),
        out_shape=jax.ShapeDtypeStruct((B, N, KNN), jnp.int32),
    )(srcp, srcpT)


def kernel(points, features):
    feats = jnp.squeeze(features, -1)
    B, C, N = feats.shape
    idx = _knn_indices(points)[:, :, 1:]
    K1 = KNN - 1
    idx_flat = idx.reshape(B, 1, N * K1)
    idx_flat = jnp.broadcast_to(idx_flat, (B, C, N * K1))
    neighbor = jnp.take_along_axis(feats, idx_flat, axis=-1).reshape(B, C, N, K1)
    center = jnp.broadcast_to(feats[..., None], (B, C, N, K1))
    return jnp.concatenate((center, neighbor - center), axis=1)
